# baseline (speedup 1.0000x reference)
import functools

import jax
import jax.numpy as jnp
from jax.experimental import pallas as pl
from jax.experimental.pallas import tpu as pltpu
from jax.experimental.pallas import tpu_sc as plsc

_ROWS = 4
_L = 16


def kernel(x, pos_table):
    B, T, D = x.shape
    mesh = plsc.VectorSubcoreMesh(core_axis_name="c", subcore_axis_name="s")

    @functools.partial(
        pl.kernel,
        out_type=jax.ShapeDtypeStruct(x.shape, x.dtype),
        mesh=mesh,
    )
    def run(x_hbm, p_hbm, o_hbm):
        def body(x_v, p_v, o_v):
            @pl.loop(0, _ROWS)
            def _row(r):
                @pl.loop(0, D, step=_L)
                def _col(j):
                    pv = p_v[r, pl.ds(j, _L)]
                    for b in range(B):
                        o_v[b, r, pl.ds(j, _L)] = x_v[b, r, pl.ds(j, _L)] + pv

        pltpu.emit_pipeline(
            body,
            grid=(T // _ROWS,),
            in_specs=[
                pl.BlockSpec((B, _ROWS, D), lambda t: (0, t, 0)),
                pl.BlockSpec((_ROWS, D), lambda t: (t, 0)),
            ],
            out_specs=[pl.BlockSpec((B, _ROWS, D), lambda t: (0, t, 0))],
            core_axis_name=("c", "s"),
            dimension_semantics=(pltpu.PARALLEL,),
        )(x_hbm, p_hbm, o_hbm)

    return run(x, pos_table)


# speedup vs baseline: 2.9259x; 2.9259x over previous
import functools

import jax
import jax.numpy as jnp
from jax.experimental import pallas as pl
from jax.experimental.pallas import tpu as pltpu
from jax.experimental.pallas import tpu_sc as plsc

_ROWS = 4
_L = 16
_UNROLL = 8


def kernel(x, pos_table):
    B, T, D = x.shape
    mesh = plsc.VectorSubcoreMesh(core_axis_name="c", subcore_axis_name="s")

    @functools.partial(
        pl.kernel,
        out_type=jax.ShapeDtypeStruct(x.shape, x.dtype),
        mesh=mesh,
    )
    def run(x_hbm, p_hbm, o_hbm):
        def body(x_v, p_v, o_v):
            for r in range(_ROWS):
                @plsc.parallel_loop(0, D, step=_L, unroll=_UNROLL)
                def _(j, r=r):
                    sl = pl.ds(j, _L)
                    pv = p_v[r, sl]
                    for b in range(B):
                        o_v[b, r, sl] = x_v[b, r, sl] + pv

        pltpu.emit_pipeline(
            body,
            grid=(T // _ROWS,),
            in_specs=[
                pl.BlockSpec((B, _ROWS, D), lambda t: (0, t, 0)),
                pl.BlockSpec((_ROWS, D), lambda t: (t, 0)),
            ],
            out_specs=[pl.BlockSpec((B, _ROWS, D), lambda t: (0, t, 0))],
            core_axis_name=("c", "s"),
            dimension_semantics=(pltpu.PARALLEL,),
        )(x_hbm, p_hbm, o_hbm)

    return run(x, pos_table)
